# baseline (speedup 1.0000x reference)
import jax
import jax.numpy as jnp
from jax import lax
from jax.experimental import pallas as pl
from jax.experimental.pallas import tpu as pltpu

K_IN = 3 * 84 * 84
D = 512
KODES = 512
NS = 100
NQ = 300
BK = 2688
KT = (K_IN + BK - 1) // BK

_HI = lax.Precision.HIGHEST
_H3 = lax.Precision.HIGH


def _dot(a, b, dims, prec):
    return lax.dot_general(a, b, (dims, ((), ())), precision=prec,
                           preferred_element_type=jnp.float32)


def _dot_bf16(a, b, dims):
    return lax.dot_general(a.astype(jnp.bfloat16), b.astype(jnp.bfloat16),
                           (dims, ((), ())),
                           preferred_element_type=jnp.float32)


def _body(xs_ref, xq_ref, w_ref, b_ref, cb_ref, t_ref, out_ref, accs, accq):
    k = pl.program_id(0)

    @pl.when(k == 0)
    def _init():
        accs[...] = jnp.zeros_like(accs)
        accq[...] = jnp.zeros_like(accq)

    def _masked():
        rem = K_IN - (KT - 1) * BK
        colmask = lax.broadcasted_iota(jnp.int32, (1, BK), 1) < rem
        rowmask = lax.broadcasted_iota(jnp.int32, (BK, 1), 0) < rem
        return (jnp.where(colmask, xs_ref[...], 0.0),
                jnp.where(colmask, xq_ref[...], 0.0),
                jnp.where(rowmask, w_ref[...], 0.0))

    def _plain():
        return xs_ref[...], xq_ref[...], w_ref[...]

    xs, xq, w = lax.cond(k == KT - 1, _masked, _plain)
    accs[...] += _dot(xs, w, ((1,), (0,)), None)
    accq[...] += _dot(xq, w, ((1,), (0,)), None)

    @pl.when(k == KT - 1)
    def _epilogue():
        bias = b_ref[...]
        cb = cb_ref[...]
        ones = jnp.ones((1, D), jnp.float32)
        cn = _dot(ones, cb * cb, ((1,), (1,)), _HI)

        def quantize(z):
            zc = _dot(z, cb, ((1,), (1,)), None)
            zn = jnp.sum(z * z, axis=1, keepdims=True)
            dist = zn - 2.0 * zc + cn
            mn = jnp.min(dist, axis=1, keepdims=True)
            ii = lax.broadcasted_iota(jnp.int32, dist.shape, 1)
            idx = jnp.min(jnp.where(dist == mn, ii, KODES), axis=1,
                          keepdims=True)
            return (ii == idx).astype(jnp.float32)

        zs = accs[...] + bias
        zq = accq[...] + bias
        qs = _dot(quantize(zs), cb, ((1,), (0,)), _HI)
        qq = _dot(quantize(zq), cb, ((1,), (0,)), _HI)

        gi = lax.broadcasted_iota(jnp.int32, (20, NS), 0)
        ci = lax.broadcasted_iota(jnp.int32, (20, NS), 1)
        sel = (ci // 5 == gi).astype(jnp.float32)
        proto = _dot(sel, qs, ((1,), (0,)), _HI) / 5.0
        pn = jnp.sqrt(jnp.sum(proto * proto, axis=1, keepdims=True))
        proto_n = proto / jnp.maximum(pn, 1e-12)
        qn = jnp.sqrt(jnp.sum(qq * qq, axis=1, keepdims=True))
        xq_n = qq / jnp.maximum(qn, 1e-12)

        lg = _dot(xq_n, proto_n, ((1,), (1,)), None) * t_ref[0, 0]
        for bb in range(4):
            out_ref[bb, :, :] = lg[75 * bb:75 * (bb + 1), 5 * bb:5 * (bb + 1)]


def kernel(x_shot, x_query, enc_W, enc_b, codebook, temp):
    xs = x_shot.reshape(NS, K_IN)
    xq = x_query.reshape(NQ, K_IN)
    b2 = enc_b.reshape(1, D)
    t2 = jnp.asarray(temp, jnp.float32).reshape(1, 1)
    return pl.pallas_call(
        _body,
        grid=(KT,),
        in_specs=[
            pl.BlockSpec((NS, BK), lambda k: (0, k)),
            pl.BlockSpec((NQ, BK), lambda k: (0, k)),
            pl.BlockSpec((BK, D), lambda k: (k, 0)),
            pl.BlockSpec((1, D), lambda k: (0, 0)),
            pl.BlockSpec((KODES, D), lambda k: (0, 0)),
            pl.BlockSpec((1, 1), lambda k: (0, 0)),
        ],
        out_specs=pl.BlockSpec((4, 75, 5), lambda k: (0, 0, 0)),
        out_shape=jax.ShapeDtypeStruct((4, 75, 5), jnp.float32),
        scratch_shapes=[
            pltpu.VMEM((NS, D), jnp.float32),
            pltpu.VMEM((NQ, D), jnp.float32),
        ],
    )(xs, xq, enc_W, b2, codebook, t2)


# speedup vs baseline: 1.1488x; 1.1488x over previous
import jax
import jax.numpy as jnp
from jax import lax
from jax.experimental import pallas as pl
from jax.experimental.pallas import tpu as pltpu

D = 512
KODES = 512
NS = 100
NQ = 300
KC = 7056
BMS = 32
BMQ = 80
MT = 4

_HI = lax.Precision.HIGHEST


def _dot(a, b, dims, prec):
    return lax.dot_general(a, b, (dims, ((), ())), precision=prec,
                           preferred_element_type=jnp.float32)


def _body(xs_ref, xq_ref, w_ref, b_ref, cb_ref, t_ref, out_ref, accs, accq):
    c = pl.program_id(0)
    m = pl.program_id(1)
    step = c * MT + m

    @pl.when(step == 0)
    def _init():
        accs[...] = jnp.zeros_like(accs)
        accq[...] = jnp.zeros_like(accq)

    xs = xs_ref[...].reshape(BMS, KC)
    xq = xq_ref[...].reshape(BMQ, KC)
    w = w_ref[...]
    zs = _dot(xs, w, ((1,), (0,)), None)
    zq = _dot(xq, w, ((1,), (0,)), None)
    accs[pl.ds(m * BMS, BMS), :] += zs
    accq[pl.ds(m * BMQ, BMQ), :] += zq

    @pl.when(step == 3 * MT - 1)
    def _epilogue():
        bias = b_ref[...]
        cb = cb_ref[...]
        ones = jnp.ones((1, D), jnp.float32)
        cn = _dot(ones, cb * cb, ((1,), (1,)), _HI)

        def quantize(z):
            zc = _dot(z, cb, ((1,), (1,)), None)
            zn = jnp.sum(z * z, axis=1, keepdims=True)
            dist = zn - 2.0 * zc + cn
            mn = jnp.min(dist, axis=1, keepdims=True)
            ii = lax.broadcasted_iota(jnp.int32, dist.shape, 1)
            idx = jnp.min(jnp.where(dist == mn, ii, KODES), axis=1,
                          keepdims=True)
            return (ii == idx).astype(jnp.float32)

        zs_f = accs[...] + bias
        zq_f = accq[...] + bias
        qs = _dot(quantize(zs_f), cb, ((1,), (0,)), _HI)
        qq = _dot(quantize(zq_f), cb, ((1,), (0,)), _HI)

        gi = lax.broadcasted_iota(jnp.int32, (20, MT * BMS), 0)
        ci = lax.broadcasted_iota(jnp.int32, (20, MT * BMS), 1)
        sel = (ci // 5 == gi).astype(jnp.float32)
        proto = _dot(sel, qs, ((1,), (0,)), _HI) / 5.0
        pn = jnp.sqrt(jnp.sum(proto * proto, axis=1, keepdims=True))
        proto_n = proto / jnp.maximum(pn, 1e-12)
        qn = jnp.sqrt(jnp.sum(qq * qq, axis=1, keepdims=True))
        xq_n = qq / jnp.maximum(qn, 1e-12)

        lg = _dot(xq_n, proto_n, ((1,), (1,)), None) * t_ref[0, 0]
        for bb in range(4):
            out_ref[bb, :, :] = lg[75 * bb:75 * (bb + 1), 5 * bb:5 * (bb + 1)]


def kernel(x_shot, x_query, enc_W, enc_b, codebook, temp):
    xs = x_shot.reshape(NS, 3, 84, 84)
    xq = x_query.reshape(NQ, 3, 84, 84)
    b2 = enc_b.reshape(1, D)
    t2 = jnp.asarray(temp, jnp.float32).reshape(1, 1)
    return pl.pallas_call(
        _body,
        grid=(3, MT),
        in_specs=[
            pl.BlockSpec((BMS, 1, 84, 84), lambda c, m: (m, c, 0, 0)),
            pl.BlockSpec((BMQ, 1, 84, 84), lambda c, m: (m, c, 0, 0)),
            pl.BlockSpec((KC, D), lambda c, m: (c, 0)),
            pl.BlockSpec((1, D), lambda c, m: (0, 0)),
            pl.BlockSpec((KODES, D), lambda c, m: (0, 0)),
            pl.BlockSpec((1, 1), lambda c, m: (0, 0)),
        ],
        out_specs=pl.BlockSpec((4, 75, 5), lambda c, m: (0, 0, 0)),
        out_shape=jax.ShapeDtypeStruct((4, 75, 5), jnp.float32),
        scratch_shapes=[
            pltpu.VMEM((MT * BMS, D), jnp.float32),
            pltpu.VMEM((MT * BMQ, D), jnp.float32),
        ],
    )(xs, xq, enc_W, b2, codebook, t2)


# speedup vs baseline: 3.1319x; 2.7263x over previous
import jax
import jax.numpy as jnp
from jax import lax
from jax.experimental import pallas as pl
from jax.experimental.pallas import tpu as pltpu

D = 512
KODES = 512
KH = 3528
BQ = 26
NQP = 3 * BQ * 4
NSP = 100

_HI = lax.Precision.HIGHEST


def _dot(a, b, dims, prec):
    return lax.dot_general(a, b, (dims, ((), ())), precision=prec,
                           preferred_element_type=jnp.float32)


def _unscramble(v):
    g = v.shape[0]
    v = v.reshape(g, 21, 2, 4, 84)
    v = v.transpose(0, 3, 1, 2, 4)
    return v.reshape(g * 4, KH)


def _body(xs_ref, xq_ref, w_ref, b_ref, cb_ref, t_ref, out_ref, accs, accq):
    c = pl.program_id(0)
    h = pl.program_id(1)
    m = pl.program_id(2)
    step = (c * 2 + h) * 3 + m

    @pl.when(step == 0)
    def _init():
        accs[...] = jnp.zeros_like(accs)
        accq[...] = jnp.zeros_like(accq)

    w = w_ref[...]

    @pl.when(m == 0)
    def _shots():
        xs = _unscramble(xs_ref[...].reshape(25, 21, 8, 84))
        accs[...] += _dot(xs, w, ((1,), (0,)), None)

    xq = _unscramble(xq_ref[...].reshape(BQ, 21, 8, 84))
    accq[pl.ds(m * BQ * 4, BQ * 4), :] += _dot(xq, w, ((1,), (0,)), None)

    @pl.when(step == 17)
    def _epilogue():
        bias = b_ref[...]
        cb = cb_ref[...]
        ones = jnp.ones((1, D), jnp.float32)
        cn = _dot(ones, cb * cb, ((1,), (1,)), _HI)

        def quantize(z):
            zc = _dot(z, cb, ((1,), (1,)), None)
            zn = jnp.sum(z * z, axis=1, keepdims=True)
            dist = zn - 2.0 * zc + cn
            mn = jnp.min(dist, axis=1, keepdims=True)
            ii = lax.broadcasted_iota(jnp.int32, dist.shape, 1)
            idx = jnp.min(jnp.where(dist == mn, ii, KODES), axis=1,
                          keepdims=True)
            return (ii == idx).astype(jnp.float32)

        zs_f = accs[...] + bias
        zq_f = accq[...] + bias
        qs = _dot(quantize(zs_f), cb, ((1,), (0,)), _HI)
        qq = _dot(quantize(zq_f), cb, ((1,), (0,)), _HI)

        gi = lax.broadcasted_iota(jnp.int32, (20, NSP), 0)
        ri = lax.broadcasted_iota(jnp.int32, (20, NSP), 1)
        sel = ((ri % 4) * 5 + ri // 20 == gi).astype(jnp.float32)
        proto = _dot(sel, qs, ((1,), (0,)), _HI) / 5.0
        pn = jnp.sqrt(jnp.sum(proto * proto, axis=1, keepdims=True))
        proto_n = proto / jnp.maximum(pn, 1e-12)
        qn = jnp.sqrt(jnp.sum(qq * qq, axis=1, keepdims=True))
        xq_n = qq / jnp.maximum(qn, 1e-12)

        lg = _dot(xq_n, proto_n, ((1,), (1,)), None) * t_ref[0, 0]

        oi = lax.broadcasted_iota(jnp.int32, (300, NQP), 0)
        rj = lax.broadcasted_iota(jnp.int32, (300, NQP), 1)
        qof = 26 * (rj // 104) + (rj % 104) // 4
        bof = rj % 4
        perm = ((oi // 75 == bof) & (oi % 75 == qof)).astype(jnp.float32)
        lgo = _dot(perm, lg, ((1,), (0,)), _HI)
        for bb in range(4):
            out_ref[bb, :, :] = lgo[75 * bb:75 * (bb + 1),
                                    5 * bb:5 * (bb + 1)]


def kernel(x_shot, x_query, enc_W, enc_b, codebook, temp):
    xs = x_shot.transpose(1, 2, 3, 4, 0, 5).reshape(5, 5, 3, 42, 8, 84)
    xq = x_query.transpose(1, 2, 3, 0, 4).reshape(75, 3, 42, 8, 84)
    b2 = enc_b.reshape(1, D)
    t2 = jnp.asarray(temp, jnp.float32).reshape(1, 1)
    return pl.pallas_call(
        _body,
        grid=(3, 2, 3),
        in_specs=[
            pl.BlockSpec((5, 5, 1, 21, 8, 84),
                         lambda c, h, m: (0, 0, c, h, 0, 0)),
            pl.BlockSpec((BQ, 1, 21, 8, 84),
                         lambda c, h, m: (m, c, h, 0, 0)),
            pl.BlockSpec((KH, D), lambda c, h, m: (c * 2 + h, 0)),
            pl.BlockSpec((1, D), lambda c, h, m: (0, 0)),
            pl.BlockSpec((KODES, D), lambda c, h, m: (0, 0)),
            pl.BlockSpec((1, 1), lambda c, h, m: (0, 0)),
        ],
        out_specs=pl.BlockSpec((4, 75, 5), lambda c, h, m: (0, 0, 0)),
        out_shape=jax.ShapeDtypeStruct((4, 75, 5), jnp.float32),
        scratch_shapes=[
            pltpu.VMEM((NSP, D), jnp.float32),
            pltpu.VMEM((NQP, D), jnp.float32),
        ],
    )(xs, xq, enc_W, b2, codebook, t2)


# speedup vs baseline: 3.4398x; 1.0983x over previous
import jax
import jax.numpy as jnp
from jax import lax
from jax.experimental import pallas as pl
from jax.experimental.pallas import tpu as pltpu

D = 512
KODES = 512
KH = 3528
BQ = 38
NQP = 2 * BQ * 4
NSP = 100

_HI = lax.Precision.HIGHEST


def _dot(a, b, dims, prec):
    return lax.dot_general(a, b, (dims, ((), ())), precision=prec,
                           preferred_element_type=jnp.float32)


def _unscramble(v):
    g = v.shape[0]
    v = v.reshape(g, 21, 2, 4, 84)
    v = v.transpose(0, 3, 1, 2, 4)
    return v.reshape(g * 4, KH)


def _body(xs_ref, xq_ref, w_ref, b_ref, cb_ref, t_ref, out_ref, accs, accq):
    c = pl.program_id(0)
    h = pl.program_id(1)
    m = pl.program_id(2)
    step = (c * 2 + h) * 2 + m

    @pl.when(step == 0)
    def _init():
        accs[...] = jnp.zeros_like(accs)
        accq[...] = jnp.zeros_like(accq)

    w = w_ref[...]

    @pl.when(m == 0)
    def _shots():
        xs = _unscramble(xs_ref[...].reshape(25, 21, 8, 84))
        accs[...] += _dot(xs, w, ((1,), (0,)), None)

    xq = _unscramble(xq_ref[...].reshape(BQ, 21, 8, 84))
    accq[pl.ds(m * BQ * 4, BQ * 4), :] += _dot(xq, w, ((1,), (0,)), None)

    @pl.when(step == 11)
    def _epilogue():
        bias = b_ref[...]
        cb = cb_ref[...]
        ones = jnp.ones((1, D), jnp.float32)
        cn = _dot(ones, cb * cb, ((1,), (1,)), _HI)

        def quantize(z):
            zc = _dot(z, cb, ((1,), (1,)), None)
            zn = jnp.sum(z * z, axis=1, keepdims=True)
            dist = zn - 2.0 * zc + cn
            mn = jnp.min(dist, axis=1, keepdims=True)
            ii = lax.broadcasted_iota(jnp.int32, dist.shape, 1)
            idx = jnp.min(jnp.where(dist == mn, ii, KODES), axis=1,
                          keepdims=True)
            return (ii == idx).astype(jnp.float32)

        zs_f = accs[...] + bias
        zq_f = accq[...] + bias
        qs = _dot(quantize(zs_f), cb, ((1,), (0,)), _HI)
        qq = _dot(quantize(zq_f), cb, ((1,), (0,)), _HI)

        gi = lax.broadcasted_iota(jnp.int32, (20, NSP), 0)
        ri = lax.broadcasted_iota(jnp.int32, (20, NSP), 1)
        sel = ((ri % 4) * 5 + ri // 20 == gi).astype(jnp.float32)
        proto = _dot(sel, qs, ((1,), (0,)), _HI) / 5.0
        pn = jnp.sqrt(jnp.sum(proto * proto, axis=1, keepdims=True))
        proto_n = proto / jnp.maximum(pn, 1e-12)
        qn = jnp.sqrt(jnp.sum(qq * qq, axis=1, keepdims=True))
        xq_n = qq / jnp.maximum(qn, 1e-12)

        lg = _dot(xq_n, proto_n, ((1,), (1,)), None) * t_ref[0, 0]

        oi = lax.broadcasted_iota(jnp.int32, (300, NQP), 0)
        rj = lax.broadcasted_iota(jnp.int32, (300, NQP), 1)
        qof = 38 * (rj // 152) + (rj % 152) // 4
        bof = rj % 4
        perm = ((oi // 75 == bof) & (oi % 75 == qof)).astype(jnp.float32)
        lgo = _dot(perm, lg, ((1,), (0,)), _HI)
        for bb in range(4):
            out_ref[bb, :, :] = lgo[75 * bb:75 * (bb + 1),
                                    5 * bb:5 * (bb + 1)]


def kernel(x_shot, x_query, enc_W, enc_b, codebook, temp):
    xs = x_shot.transpose(1, 2, 3, 4, 0, 5).reshape(5, 5, 3, 42, 8, 84)
    xq = x_query.transpose(1, 2, 3, 0, 4).reshape(75, 3, 42, 8, 84)
    b2 = enc_b.reshape(1, D)
    t2 = jnp.asarray(temp, jnp.float32).reshape(1, 1)
    return pl.pallas_call(
        _body,
        grid=(3, 2, 2),
        in_specs=[
            pl.BlockSpec((5, 5, 1, 21, 8, 84),
                         lambda c, h, m: (0, 0, c, h, 0, 0)),
            pl.BlockSpec((BQ, 1, 21, 8, 84),
                         lambda c, h, m: (m, c, h, 0, 0)),
            pl.BlockSpec((KH, D), lambda c, h, m: (c * 2 + h, 0)),
            pl.BlockSpec((1, D), lambda c, h, m: (0, 0)),
            pl.BlockSpec((KODES, D), lambda c, h, m: (0, 0)),
            pl.BlockSpec((1, 1), lambda c, h, m: (0, 0)),
        ],
        out_specs=pl.BlockSpec((4, 75, 5), lambda c, h, m: (0, 0, 0)),
        out_shape=jax.ShapeDtypeStruct((4, 75, 5), jnp.float32),
        scratch_shapes=[
            pltpu.VMEM((NSP, D), jnp.float32),
            pltpu.VMEM((NQP, D), jnp.float32),
        ],
    )(xs, xq, enc_W, b2, codebook, t2)


# speedup vs baseline: 4.6608x; 1.3549x over previous
import jax
import jax.numpy as jnp
from jax import lax
from jax.experimental import pallas as pl
from jax.experimental.pallas import tpu as pltpu

D = 512
KODES = 512
KH = 3528
BQ = 76
NQP = BQ * 4
NSP = 100

_HI = lax.Precision.HIGHEST


def _dot(a, b, dims, prec):
    return lax.dot_general(a, b, (dims, ((), ())), precision=prec,
                           preferred_element_type=jnp.float32)


def _unscramble(v):
    g = v.shape[0]
    v = v.reshape(g, 21, 2, 4, 84)
    v = v.transpose(0, 3, 1, 2, 4)
    return v.reshape(g * 4, KH)


def _body(xs_ref, xq_ref, w_ref, b_ref, cb_ref, t_ref, out_ref, accs, accq):
    c = pl.program_id(0)
    h = pl.program_id(1)
    step = c * 2 + h

    @pl.when(step == 0)
    def _init():
        accs[...] = jnp.zeros_like(accs)
        accq[...] = jnp.zeros_like(accq)

    w = w_ref[...]

    xs = _unscramble(xs_ref[...].reshape(25, 21, 8, 84))
    accs[...] += _dot(xs, w, ((1,), (0,)), None)

    xq = _unscramble(xq_ref[...].reshape(BQ, 21, 8, 84))
    accq[...] += _dot(xq, w, ((1,), (0,)), None)

    @pl.when(step == 5)
    def _epilogue():
        bias = b_ref[...]
        cb = cb_ref[...]
        ones = jnp.ones((1, D), jnp.float32)
        cn = _dot(ones, cb * cb, ((1,), (1,)), _HI)

        def quantize(z):
            zc = _dot(z, cb, ((1,), (1,)), None)
            zn = jnp.sum(z * z, axis=1, keepdims=True)
            dist = zn - 2.0 * zc + cn
            mn = jnp.min(dist, axis=1, keepdims=True)
            ii = lax.broadcasted_iota(jnp.int32, dist.shape, 1)
            idx = jnp.min(jnp.where(dist == mn, ii, KODES), axis=1,
                          keepdims=True)
            return (ii == idx).astype(jnp.float32)

        zs_f = accs[...] + bias
        zq_f = accq[...] + bias
        qs = _dot(quantize(zs_f), cb, ((1,), (0,)), _HI)
        qq = _dot(quantize(zq_f), cb, ((1,), (0,)), _HI)

        gi = lax.broadcasted_iota(jnp.int32, (20, NSP), 0)
        ri = lax.broadcasted_iota(jnp.int32, (20, NSP), 1)
        sel = ((ri % 4) * 5 + ri // 20 == gi).astype(jnp.float32)
        proto = _dot(sel, qs, ((1,), (0,)), _HI) / 5.0
        pn = jnp.sqrt(jnp.sum(proto * proto, axis=1, keepdims=True))
        proto_n = proto / jnp.maximum(pn, 1e-12)
        qn = jnp.sqrt(jnp.sum(qq * qq, axis=1, keepdims=True))
        xq_n = qq / jnp.maximum(qn, 1e-12)

        lg = _dot(xq_n, proto_n, ((1,), (1,)), None) * t_ref[0, 0]

        oi = lax.broadcasted_iota(jnp.int32, (300, NQP), 0)
        rj = lax.broadcasted_iota(jnp.int32, (300, NQP), 1)
        qof = rj // 4
        bof = rj % 4
        perm = ((oi // 75 == bof) & (oi % 75 == qof)).astype(jnp.float32)
        lgo = _dot(perm, lg, ((1,), (0,)), _HI)
        for bb in range(4):
            out_ref[bb, :, :] = lgo[75 * bb:75 * (bb + 1),
                                    5 * bb:5 * (bb + 1)]


def kernel(x_shot, x_query, enc_W, enc_b, codebook, temp):
    xs = x_shot.transpose(1, 2, 3, 4, 0, 5).reshape(5, 5, 3, 42, 8, 84)
    xq = x_query.transpose(1, 2, 3, 0, 4).reshape(75, 3, 42, 8, 84)
    b2 = enc_b.reshape(1, D)
    t2 = jnp.asarray(temp, jnp.float32).reshape(1, 1)
    return pl.pallas_call(
        _body,
        grid=(3, 2),
        in_specs=[
            pl.BlockSpec((5, 5, 1, 21, 8, 84),
                         lambda c, h: (0, 0, c, h, 0, 0)),
            pl.BlockSpec((BQ, 1, 21, 8, 84),
                         lambda c, h: (0, c, h, 0, 0)),
            pl.BlockSpec((KH, D), lambda c, h: (c * 2 + h, 0)),
            pl.BlockSpec((1, D), lambda c, h: (0, 0)),
            pl.BlockSpec((KODES, D), lambda c, h: (0, 0)),
            pl.BlockSpec((1, 1), lambda c, h: (0, 0)),
        ],
        out_specs=pl.BlockSpec((4, 75, 5), lambda c, h: (0, 0, 0)),
        out_shape=jax.ShapeDtypeStruct((4, 75, 5), jnp.float32),
        scratch_shapes=[
            pltpu.VMEM((NSP, D), jnp.float32),
            pltpu.VMEM((NQP, D), jnp.float32),
        ],
    )(xs, xq, enc_W, b2, codebook, t2)
